# baseline (speedup 1.0000x reference)
import functools

import jax
import jax.numpy as jnp
from jax import lax
from jax.experimental import pallas as pl
from jax.experimental.pallas import tpu as pltpu
from jax.experimental.pallas import tpu_sc as plsc

NROWS = 16384
NCOLS = 4096
LANES = 16
NCORES = 2
NSUBCORES = 16
NWORKERS = NCORES * NSUBCORES
ROWS_PER_WORKER = NROWS // NWORKERS
BLK = 4
NBUF = 4
NBLK = ROWS_PER_WORKER // BLK
NGRP = NBLK // NBUF
NSEG = NCOLS // LANES
BLK_ELEMS = BLK * NCOLS


def _cumsum_body(x_hbm, out_hbm, *refs):
    bufs = refs[:NBUF]
    in_sems = refs[NBUF:2 * NBUF]
    out_sems = refs[2 * NBUF:3 * NBUF]

    c = lax.axis_index("c")
    s = lax.axis_index("s")
    wid = s * NCORES + c
    base = wid * ROWS_PER_WORKER * NCOLS

    def in_copy(b, p):
        return pltpu.make_async_copy(
            x_hbm.at[pl.ds(base + b * BLK_ELEMS, BLK_ELEMS)], bufs[p],
            in_sems[p]
        )

    def out_copy(b, p):
        return pltpu.make_async_copy(
            bufs[p], out_hbm.at[pl.ds(base + b * BLK_ELEMS, BLK_ELEMS)],
            out_sems[p]
        )

    def compute(buf):
        zeros = tuple(jnp.float32(0.0) for _ in range(BLK))

        @plsc.parallel_loop(0, NSEG, 1, unroll=16, carry=zeros)
        def _(j, carries):
            new = []
            for r in range(BLK):
                seg = buf[pl.ds(r * NCOLS + j * LANES, LANES)]
                out = plsc.cumsum(seg) + carries[r]
                buf[pl.ds(r * NCOLS + j * LANES, LANES)] = out
                new.append(out[LANES - 1])
            return tuple(new)

    for p in range(NBUF):
        in_copy(p, p).start()

    def grp_body(g, carry):
        for p in range(NBUF):
            b = g * NBUF + p
            q = (p + NBUF - 1) % NBUF

            @pl.when(jnp.logical_and(b >= 1, b <= NBLK - NBUF))
            def _():
                out_copy(b - 1, q).wait()
                in_copy(b + NBUF - 1, q).start()

            in_copy(b, p).wait()
            compute(bufs[p])
            out_copy(b, p).start()
        return carry

    lax.fori_loop(0, NGRP, grp_body, 0)

    for q in range(NBUF):
        out_copy(NBLK - NBUF + q, q).wait()


@jax.jit
def kernel(x):
    mesh = plsc.VectorSubcoreMesh(core_axis_name="c", subcore_axis_name="s")
    run = functools.partial(
        pl.kernel,
        mesh=mesh,
        out_type=jax.ShapeDtypeStruct((NROWS * NCOLS,), jnp.float32),
        scratch_types=(
            [pltpu.VMEM((BLK_ELEMS,), jnp.float32) for _ in range(NBUF)]
            + [pltpu.SemaphoreType.DMA for _ in range(2 * NBUF)]
        ),
        compiler_params=pltpu.CompilerParams(needs_layout_passes=False),
    )(_cumsum_body)
    return run(x.reshape(NROWS * NCOLS)).reshape(NROWS, NCOLS)


# speedup vs baseline: 3.1663x; 3.1663x over previous
import functools

import jax
import jax.numpy as jnp
from jax import lax
from jax.experimental import pallas as pl
from jax.experimental.pallas import tpu as pltpu
from jax.experimental.pallas import tpu_sc as plsc

NROWS = 16384
NCOLS = 4096
LANES = 16
NCORES = 2
NSUBCORES = 16
NWORKERS = NCORES * NSUBCORES
ROWS_PER_WORKER = NROWS // NWORKERS
BLK = 4
NBUF = 4
NBLK = ROWS_PER_WORKER // BLK
NGRP = NBLK // NBUF
NSEG = NCOLS // LANES


def _cumsum_body(x_hbm, out_hbm, *refs):
    bufs = refs[:NBUF]
    in_sems = refs[NBUF:2 * NBUF]
    out_sems = refs[2 * NBUF:3 * NBUF]

    c = lax.axis_index("c")
    s = lax.axis_index("s")
    wid = s * NCORES + c
    base = wid * ROWS_PER_WORKER

    def in_copy(b, p):
        return pltpu.make_async_copy(
            x_hbm.at[pl.ds(base + b * BLK, BLK)], bufs[p], in_sems[p]
        )

    def out_copy(b, p):
        return pltpu.make_async_copy(
            bufs[p], out_hbm.at[pl.ds(base + b * BLK, BLK)], out_sems[p]
        )

    def compute(buf):
        zeros = tuple(jnp.float32(0.0) for _ in range(BLK))

        @plsc.parallel_loop(0, NSEG, 1, unroll=32, carry=zeros)
        def _(j, carries):
            new = []
            for r in range(BLK):
                seg = buf[r, pl.ds(j * LANES, LANES)]
                out = plsc.cumsum(seg) + carries[r]
                buf[r, pl.ds(j * LANES, LANES)] = out
                new.append(out[LANES - 1])
            return tuple(new)

    for p in range(NBUF):
        in_copy(p, p).start()

    def grp_body(g, carry):
        for p in range(NBUF):
            b = g * NBUF + p
            q = (p + NBUF - 1) % NBUF

            @pl.when(jnp.logical_and(b >= 1, b <= NBLK - NBUF))
            def _():
                out_copy(b - 1, q).wait()
                in_copy(b + NBUF - 1, q).start()

            in_copy(b, p).wait()
            compute(bufs[p])
            out_copy(b, p).start()
        return carry

    lax.fori_loop(0, NGRP, grp_body, 0)

    for q in range(NBUF):
        out_copy(NBLK - NBUF + q, q).wait()


@jax.jit
def kernel(x):
    mesh = plsc.VectorSubcoreMesh(core_axis_name="c", subcore_axis_name="s")
    run = functools.partial(
        pl.kernel,
        mesh=mesh,
        out_type=jax.ShapeDtypeStruct((NROWS, NCOLS), jnp.float32),
        scratch_types=(
            [pltpu.VMEM((BLK, NCOLS), jnp.float32) for _ in range(NBUF)]
            + [pltpu.SemaphoreType.DMA for _ in range(2 * NBUF)]
        ),
        compiler_params=pltpu.CompilerParams(needs_layout_passes=False),
    )(_cumsum_body)
    return run(x)


# speedup vs baseline: 3.2831x; 1.0369x over previous
import functools

import jax
import jax.numpy as jnp
from jax import lax
from jax.experimental import pallas as pl
from jax.experimental.pallas import tpu as pltpu
from jax.experimental.pallas import tpu_sc as plsc

NROWS = 16384
NCOLS = 4096
LANES = 16
NCORES = 2
NSUBCORES = 16
NWORKERS = NCORES * NSUBCORES
ROWS_PER_WORKER = NROWS // NWORKERS
BLK = 4
NBUF = 4
NBLK = ROWS_PER_WORKER // BLK
NGRP = NBLK // NBUF
NSEG = NCOLS // LANES


def _cumsum_body(x_hbm, out_hbm, *refs):
    bufs = refs[:NBUF]
    in_sems = refs[NBUF:2 * NBUF]
    out_sems = refs[2 * NBUF:3 * NBUF]

    c = lax.axis_index("c")
    s = lax.axis_index("s")
    wid = s * NCORES + c
    base = wid * ROWS_PER_WORKER

    def in_copy(b, p):
        return pltpu.make_async_copy(
            x_hbm.at[pl.ds(base + b * BLK, BLK)], bufs[p], in_sems[p]
        )

    def out_copy(b, p):
        return pltpu.make_async_copy(
            bufs[p], out_hbm.at[pl.ds(base + b * BLK, BLK)], out_sems[p]
        )

    def compute(buf):
        zeros = tuple(jnp.float32(0.0) for _ in range(BLK))

        @plsc.parallel_loop(0, NSEG, 1, unroll=64, carry=zeros)
        def _(j, carries):
            new = []
            for r in range(BLK):
                seg = buf[r, pl.ds(j * LANES, LANES)]
                out = plsc.cumsum(seg) + carries[r]
                buf[r, pl.ds(j * LANES, LANES)] = out
                new.append(out[LANES - 1])
            return tuple(new)

    for p in range(NBUF):
        in_copy(p, p).start()

    def grp_body(g, carry):
        for p in range(NBUF):
            b = g * NBUF + p
            q = (p + NBUF - 1) % NBUF

            @pl.when(jnp.logical_and(b >= 1, b <= NBLK - NBUF))
            def _():
                out_copy(b - 1, q).wait()
                in_copy(b + NBUF - 1, q).start()

            in_copy(b, p).wait()
            compute(bufs[p])
            out_copy(b, p).start()
        return carry

    lax.fori_loop(0, NGRP, grp_body, 0)

    for q in range(NBUF):
        out_copy(NBLK - NBUF + q, q).wait()


@jax.jit
def kernel(x):
    mesh = plsc.VectorSubcoreMesh(core_axis_name="c", subcore_axis_name="s")
    run = functools.partial(
        pl.kernel,
        mesh=mesh,
        out_type=jax.ShapeDtypeStruct((NROWS, NCOLS), jnp.float32),
        scratch_types=(
            [pltpu.VMEM((BLK, NCOLS), jnp.float32) for _ in range(NBUF)]
            + [pltpu.SemaphoreType.DMA for _ in range(2 * NBUF)]
        ),
        compiler_params=pltpu.CompilerParams(needs_layout_passes=False),
    )(_cumsum_body)
    return run(x)
